# baseline (speedup 1.0000x reference)
import jax
import jax.numpy as jnp
from jax import lax
from jax.experimental import pallas as pl
from jax.experimental.pallas import tpu as pltpu
from jax.experimental.pallas import tpu_sc as plsc

_W = 128


def kernel(ids, table):
    B, S = ids.shape
    V, D = table.shape
    assert (B, S, D) == (16384, 200, 32)

    ids = ids.astype(jnp.int32)
    i5 = ids.T.reshape(S // 8, 8, B // _W, _W).transpose(0, 2, 1, 3)

    mesh = plsc.VectorSubcoreMesh(core_axis_name="c", subcore_axis_name="s")
    cp = pltpu.CompilerParams(
        use_tc_tiling_on_sc=False, needs_layout_passes=False
    )

    @jax.jit
    def run(table_arr, idx_arr):
        @pl.kernel(
            out_type=jax.ShapeDtypeStruct((S, D // 8, B // _W, 8, _W),
                                          table_arr.dtype),
            mesh=mesh,
            compiler_params=cp,
            scratch_types=[pltpu.VMEM((_W, D), table_arr.dtype)],
        )
        def k(table_hbm, i_hbm, o_hbm, r_vmem):
            lane = lax.iota(jnp.int32, 16)

            def body(i_ref, o_ref):
                pltpu.sync_copy(table_hbm.at[i_ref.at[0, 0, 0]], r_vmem)
                for e in range(D):
                    col = jnp.full((16,), e, jnp.int32)
                    for c in range(_W // 16):
                        vals = plsc.load_gather(r_vmem, [lane + 16 * c, col])
                        o_ref[0, e // 8, 0, e % 8, pl.ds(16 * c, 16)] = vals

            pltpu.emit_pipeline(
                body,
                grid=(S // 8, B // _W, 8),
                in_specs=[pl.BlockSpec((1, 1, 1, _W),
                                       index_map=lambda st, bt, ss: (st, bt, ss, 0))],
                out_specs=[pl.BlockSpec((1, D // 8, 1, 8, _W),
                                        index_map=lambda st, bt, ss: (st * 8 + ss, 0, bt, 0, 0))],
                core_axis_name=("c", "s"),
                dimension_semantics=(pltpu.PARALLEL, pltpu.PARALLEL, pltpu.PARALLEL),
            )(i_hbm, o_hbm)

        return k(table_arr, idx_arr)

    f = run(table, i5)
    return f.transpose(2, 4, 0, 1, 3).reshape(B, S, D)


# speedup vs baseline: 1.4056x; 1.4056x over previous
import jax
import jax.numpy as jnp
from jax import lax
from jax.experimental import pallas as pl
from jax.experimental.pallas import tpu as pltpu
from jax.experimental.pallas import tpu_sc as plsc

_W = 128


def kernel(ids, table):
    B, S = ids.shape
    V, D = table.shape
    assert (B, S, D) == (16384, 200, 32)

    ids = ids.astype(jnp.int32)
    i5 = ids.T.reshape(S // 8, 8, B // _W, _W).transpose(0, 2, 1, 3)

    mesh = plsc.VectorSubcoreMesh(core_axis_name="c", subcore_axis_name="s")
    cp = pltpu.CompilerParams(
        use_tc_tiling_on_sc=False, needs_layout_passes=False
    )

    @jax.jit
    def run(table_arr, idx_arr):
        @pl.kernel(
            out_type=jax.ShapeDtypeStruct((S, D // 8, B // _W, 8, _W),
                                          table_arr.dtype),
            mesh=mesh,
            compiler_params=cp,
            scratch_types=[pltpu.VMEM((_W, D), table_arr.dtype)],
        )
        def k(table_hbm, i_hbm, o_hbm, r_vmem):
            lane = lax.iota(jnp.int32, 16)
            rows = [lane + 16 * c for c in range(_W // 16)]
            cols = [jnp.full((16,), e, jnp.int32) for e in range(D)]

            def body(i_ref, o_ref):
                pltpu.sync_copy(table_hbm.at[i_ref.at[0, 0, 0]], r_vmem)
                for c in range(_W // 16):
                    vals = [plsc.load_gather(r_vmem, [rows[c], cols[e]])
                            for e in range(D)]
                    for e in range(D):
                        o_ref[0, e // 8, 0, e % 8, pl.ds(16 * c, 16)] = vals[e]

            pltpu.emit_pipeline(
                body,
                grid=(S // 8, B // _W, 8),
                in_specs=[pl.BlockSpec((1, 1, 1, _W),
                                       index_map=lambda st, bt, ss: (st, bt, ss, 0))],
                out_specs=[pl.BlockSpec((1, D // 8, 1, 8, _W),
                                        index_map=lambda st, bt, ss: (st * 8 + ss, 0, bt, 0, 0))],
                core_axis_name=("c", "s"),
                dimension_semantics=(pltpu.PARALLEL, pltpu.PARALLEL, pltpu.PARALLEL),
            )(i_hbm, o_hbm)

        return k(table_arr, idx_arr)

    f = run(table, i5)
    return f.transpose(2, 4, 0, 1, 3).reshape(B, S, D)
